# baseline (speedup 1.0000x reference)
import jax
import jax.numpy as jnp
from jax.experimental import pallas as pl
from jax.experimental.pallas import tpu as pltpu

N = 20000
DET = 100
SCORE_THRESH = 0.05
NMS_THRESH = 0.5
NEG = -1e9
BIGF = 3e7

ROWS = 160
LANES = 128
K = 6


def _iou_parts(b, x1, y1, x2, y2):
    bx1, by1, bx2, by2 = b
    inter = (jnp.maximum(jnp.minimum(bx2, x2) - jnp.maximum(bx1, x1), 0.0)
             * jnp.maximum(jnp.minimum(by2, y2) - jnp.maximum(by1, y1), 0.0))
    barea = (bx2 - bx1) * (by2 - by1)
    area = (x2 - x1) * (y2 - y1)
    return inter / (barea + area - inter + 1e-9)


def _nms_body(x1_ref, y1_ref, x2_ref, y2_ref, s_ref, out_ref, sc_ref, st_ref):
    lane_f = jax.lax.broadcasted_iota(jnp.int32, (1, LANES), 1).astype(
        jnp.float32)
    lane_i = jax.lax.broadcasted_iota(jnp.int32, (1, LANES), 1)
    row160_f = jax.lax.broadcasted_iota(jnp.int32, (ROWS, 1), 0).astype(
        jnp.float32)
    flat_iota_f = (
        jax.lax.broadcasted_iota(jnp.int32, (ROWS, LANES), 0) * LANES
        + jax.lax.broadcasted_iota(jnp.int32, (ROWS, LANES), 1)
    ).astype(jnp.float32)

    out_ref[...] = jnp.zeros((DET, LANES), jnp.float32)
    sc_ref[...] = jnp.where(s_ref[...] > SCORE_THRESH, s_ref[...], NEG)
    st_ref[0] = 0
    st_ref[1] = 1

    def rnd(r, _):
        @pl.when((st_ref[0] < DET) & (st_ref[1] == 1))
        def _():
            p0 = st_ref[0]
            sc = sc_ref[...]
            x1 = x1_ref[...]
            y1 = y1_ref[...]
            x2 = x2_ref[...]
            y2 = y2_ref[...]

            s_lv, f_lv, pb_lv = [], [], []
            masked = sc
            for j in range(K):
                sj = jnp.max(masked, axis=0, keepdims=True)
                rowhit = jnp.min(jnp.where(masked == sj, row160_f, BIGF),
                                 axis=0, keepdims=True)
                rowsel = row160_f == rowhit
                pbj = tuple(
                    jnp.max(jnp.where(rowsel, v, -1e30), axis=0, keepdims=True)
                    for v in (x1, y1, x2, y2))
                s_lv.append(sj)
                f_lv.append(rowhit * LANES + lane_f)
                pb_lv.append(pbj)
                if j < K - 1:
                    masked = jnp.where(rowsel, NEG, masked)

            consumed = jnp.zeros((1, LANES), jnp.int32)
            cand_s, cand_f = s_lv[0], f_lv[0]
            cand_pb = list(pb_lv[0])
            m = jnp.max(cand_s, axis=1, keepdims=True)
            ms, idxs, boxes = [], [], []
            for j in range(K):
                idx = jnp.min(jnp.where(cand_s == m, cand_f, BIGF), axis=1,
                              keepdims=True)
                lanewin = cand_f == idx
                boxes.append(tuple(
                    jnp.max(jnp.where(lanewin, pb, -1e30), axis=1,
                            keepdims=True) for pb in cand_pb))
                ms.append(m)
                idxs.append(idx)
                if j < K - 1:
                    consumed = consumed + lanewin.astype(jnp.int32)
                    nxt_s = jnp.full((1, LANES), NEG, jnp.float32)
                    nxt_f = jnp.full((1, LANES), BIGF, jnp.float32)
                    nxt_pb = [jnp.zeros((1, LANES), jnp.float32)
                              for _ in range(4)]
                    for lv in range(K - 1, 0, -1):
                        hit = consumed == lv
                        nxt_s = jnp.where(hit, s_lv[lv], nxt_s)
                        nxt_f = jnp.where(hit, f_lv[lv], nxt_f)
                        nxt_pb = [jnp.where(hit, pb_lv[lv][q], nxt_pb[q])
                                  for q in range(4)]
                    cand_s = jnp.where(lanewin, nxt_s, cand_s)
                    cand_f = jnp.where(lanewin, nxt_f, cand_f)
                    cand_pb = [jnp.where(lanewin, nxt_pb[q], cand_pb[q])
                               for q in range(4)]
                    m = jnp.max(cand_s, axis=1, keepdims=True)

            sels = []
            for j in range(K):
                ok = ms[j] > SCORE_THRESH
                for i in range(j):
                    compat = _iou_parts(boxes[i], *boxes[j]) <= NMS_THRESH
                    ok = ok & (jnp.logical_not(sels[i]) | compat)
                sels.append(ok)

            supp = jnp.zeros((ROWS, LANES), jnp.bool_)
            for j in range(K):
                hit = (_iou_parts(boxes[j], x1, y1, x2, y2) > NMS_THRESH) | (
                    flat_iota_f == idxs[j])
                supp = supp | (sels[j] & hit)
            sc_ref[...] = jnp.where(supp, NEG, sc)

            pj = p0
            for j in range(K):
                bx1, by1, bx2, by2 = boxes[j]
                rowv = jnp.where(
                    lane_i == 0, bx1,
                    jnp.where(lane_i == 1, by1,
                    jnp.where(lane_i == 2, bx2,
                    jnp.where(lane_i == 3, by2, ms[j]))))
                sel_s = sels[j].astype(jnp.int32)[0, 0]

                @pl.when((sel_s == 1) & (pj < DET))
                def _(rowv=rowv, pj=pj):
                    out_ref[pl.ds(pj, 1), :] = rowv

                pj = pj + sel_s
            st_ref[0] = pj
            st_ref[1] = jnp.where(ms[0][0, 0] > SCORE_THRESH, 1, 0)
        return 0

    jax.lax.fori_loop(0, DET, rnd, 0)


@jax.jit
def kernel(boxes, scores):
    pad = ROWS * LANES - N
    x1 = jnp.pad(boxes[:, 0], (0, pad)).reshape(ROWS, LANES)
    y1 = jnp.pad(boxes[:, 1], (0, pad)).reshape(ROWS, LANES)
    x2 = jnp.pad(boxes[:, 2], (0, pad)).reshape(ROWS, LANES)
    y2 = jnp.pad(boxes[:, 3], (0, pad)).reshape(ROWS, LANES)
    s = jnp.pad(scores, (0, pad)).reshape(ROWS, LANES)

    out = pl.pallas_call(
        _nms_body,
        out_shape=jax.ShapeDtypeStruct((DET, LANES), jnp.float32),
        scratch_shapes=[pltpu.VMEM((ROWS, LANES), jnp.float32),
                        pltpu.SMEM((2,), jnp.int32)],
    )(x1, y1, x2, y2, s)
    return out[:, :5]


# speedup vs baseline: 1.0581x; 1.0581x over previous
import jax
import jax.numpy as jnp
from jax.experimental import pallas as pl
from jax.experimental.pallas import tpu as pltpu

N = 20000
DET = 100
SCORE_THRESH = 0.05
NMS_THRESH = 0.5
NEG = -1e9
BIGF = 3e7

ROWS = 160
LANES = 128
K = 2


def _iou_parts(b, x1, y1, x2, y2):
    bx1, by1, bx2, by2 = b
    inter = (jnp.maximum(jnp.minimum(bx2, x2) - jnp.maximum(bx1, x1), 0.0)
             * jnp.maximum(jnp.minimum(by2, y2) - jnp.maximum(by1, y1), 0.0))
    barea = (bx2 - bx1) * (by2 - by1)
    area = (x2 - x1) * (y2 - y1)
    return inter / (barea + area - inter + 1e-9)


def _nms_body(x1_ref, y1_ref, x2_ref, y2_ref, s_ref, out_ref, sc_ref, st_ref):
    lane_f = jax.lax.broadcasted_iota(jnp.int32, (1, LANES), 1).astype(
        jnp.float32)
    lane_i = jax.lax.broadcasted_iota(jnp.int32, (1, LANES), 1)
    row160_f = jax.lax.broadcasted_iota(jnp.int32, (ROWS, 1), 0).astype(
        jnp.float32)
    flat_iota_f = (
        jax.lax.broadcasted_iota(jnp.int32, (ROWS, LANES), 0) * LANES
        + jax.lax.broadcasted_iota(jnp.int32, (ROWS, LANES), 1)
    ).astype(jnp.float32)

    out_ref[...] = jnp.zeros((DET, LANES), jnp.float32)
    sc_ref[...] = jnp.where(s_ref[...] > SCORE_THRESH, s_ref[...], NEG)
    st_ref[0] = 0
    st_ref[1] = 1

    def rnd(r, _):
        @pl.when((st_ref[0] < DET) & (st_ref[1] == 1))
        def _():
            p0 = st_ref[0]
            sc = sc_ref[...]
            x1 = x1_ref[...]
            y1 = y1_ref[...]
            x2 = x2_ref[...]
            y2 = y2_ref[...]

            s_lv, f_lv, pb_lv = [], [], []
            masked = sc
            for j in range(K):
                sj = jnp.max(masked, axis=0, keepdims=True)
                rowhit = jnp.min(jnp.where(masked == sj, row160_f, BIGF),
                                 axis=0, keepdims=True)
                rowsel = row160_f == rowhit
                pbj = tuple(
                    jnp.max(jnp.where(rowsel, v, -1e30), axis=0, keepdims=True)
                    for v in (x1, y1, x2, y2))
                s_lv.append(sj)
                f_lv.append(rowhit * LANES + lane_f)
                pb_lv.append(pbj)
                if j < K - 1:
                    masked = jnp.where(rowsel, NEG, masked)

            consumed = jnp.zeros((1, LANES), jnp.int32)
            cand_s, cand_f = s_lv[0], f_lv[0]
            cand_pb = list(pb_lv[0])
            m = jnp.max(cand_s, axis=1, keepdims=True)
            ms, idxs, boxes = [], [], []
            for j in range(K):
                idx = jnp.min(jnp.where(cand_s == m, cand_f, BIGF), axis=1,
                              keepdims=True)
                lanewin = cand_f == idx
                boxes.append(tuple(
                    jnp.max(jnp.where(lanewin, pb, -1e30), axis=1,
                            keepdims=True) for pb in cand_pb))
                ms.append(m)
                idxs.append(idx)
                if j < K - 1:
                    consumed = consumed + lanewin.astype(jnp.int32)
                    nxt_s = jnp.full((1, LANES), NEG, jnp.float32)
                    nxt_f = jnp.full((1, LANES), BIGF, jnp.float32)
                    nxt_pb = [jnp.zeros((1, LANES), jnp.float32)
                              for _ in range(4)]
                    for lv in range(K - 1, 0, -1):
                        hit = consumed == lv
                        nxt_s = jnp.where(hit, s_lv[lv], nxt_s)
                        nxt_f = jnp.where(hit, f_lv[lv], nxt_f)
                        nxt_pb = [jnp.where(hit, pb_lv[lv][q], nxt_pb[q])
                                  for q in range(4)]
                    cand_s = jnp.where(lanewin, nxt_s, cand_s)
                    cand_f = jnp.where(lanewin, nxt_f, cand_f)
                    cand_pb = [jnp.where(lanewin, nxt_pb[q], cand_pb[q])
                               for q in range(4)]
                    m = jnp.max(cand_s, axis=1, keepdims=True)

            sels = []
            for j in range(K):
                ok = ms[j] > SCORE_THRESH
                for i in range(j):
                    compat = _iou_parts(boxes[i], *boxes[j]) <= NMS_THRESH
                    ok = ok & (jnp.logical_not(sels[i]) | compat)
                sels.append(ok)

            supp = jnp.zeros((ROWS, LANES), jnp.bool_)
            for j in range(K):
                hit = (_iou_parts(boxes[j], x1, y1, x2, y2) > NMS_THRESH) | (
                    flat_iota_f == idxs[j])
                supp = supp | (sels[j] & hit)
            sc_ref[...] = jnp.where(supp, NEG, sc)

            pj = p0
            for j in range(K):
                bx1, by1, bx2, by2 = boxes[j]
                rowv = jnp.where(
                    lane_i == 0, bx1,
                    jnp.where(lane_i == 1, by1,
                    jnp.where(lane_i == 2, bx2,
                    jnp.where(lane_i == 3, by2, ms[j]))))
                sel_s = sels[j].astype(jnp.int32)[0, 0]

                @pl.when((sel_s == 1) & (pj < DET))
                def _(rowv=rowv, pj=pj):
                    out_ref[pl.ds(pj, 1), :] = rowv

                pj = pj + sel_s
            st_ref[0] = pj
            st_ref[1] = jnp.where(ms[0][0, 0] > SCORE_THRESH, 1, 0)
        return 0

    jax.lax.fori_loop(0, DET, rnd, 0)


@jax.jit
def kernel(boxes, scores):
    pad = ROWS * LANES - N
    x1 = jnp.pad(boxes[:, 0], (0, pad)).reshape(ROWS, LANES)
    y1 = jnp.pad(boxes[:, 1], (0, pad)).reshape(ROWS, LANES)
    x2 = jnp.pad(boxes[:, 2], (0, pad)).reshape(ROWS, LANES)
    y2 = jnp.pad(boxes[:, 3], (0, pad)).reshape(ROWS, LANES)
    s = jnp.pad(scores, (0, pad)).reshape(ROWS, LANES)

    out = pl.pallas_call(
        _nms_body,
        out_shape=jax.ShapeDtypeStruct((DET, LANES), jnp.float32),
        scratch_shapes=[pltpu.VMEM((ROWS, LANES), jnp.float32),
                        pltpu.SMEM((2,), jnp.int32)],
    )(x1, y1, x2, y2, s)
    return out[:, :5]


# speedup vs baseline: 1.0877x; 1.0280x over previous
import jax
import jax.numpy as jnp
from jax.experimental import pallas as pl
from jax.experimental.pallas import tpu as pltpu

N = 20000
DET = 100
SCORE_THRESH = 0.05
NMS_THRESH = 0.5
NEG = -1e9

ROWS = 160
LANES = 128
BIGF = 3e7


def _nms_body(x1_ref, y1_ref, x2_ref, y2_ref, s_ref, out_ref, sc_ref):
    lane = jax.lax.broadcasted_iota(jnp.int32, (1, LANES), 1)
    lane_f = lane.astype(jnp.float32)
    row160_f = jax.lax.broadcasted_iota(jnp.int32, (ROWS, 1), 0).astype(
        jnp.float32)
    flat_iota_f = (
        jax.lax.broadcasted_iota(jnp.int32, (ROWS, LANES), 0) * LANES
        + jax.lax.broadcasted_iota(jnp.int32, (ROWS, LANES), 1)
    ).astype(jnp.float32)

    sc0 = jnp.where(s_ref[...] > SCORE_THRESH, s_ref[...], NEG)

    def _one(i, sc):
        x1 = x1_ref[...]
        y1 = y1_ref[...]
        x2 = x2_ref[...]
        y2 = y2_ref[...]

        pls = jnp.max(sc, axis=0, keepdims=True)
        rowhit = jnp.min(jnp.where(sc == pls, row160_f, BIGF), axis=0,
                         keepdims=True)
        flat = rowhit * LANES + lane_f
        m = jnp.max(pls, axis=1, keepdims=True)
        idx = jnp.min(jnp.where(pls == m, flat, BIGF), axis=1,
                      keepdims=True)
        rowsel = row160_f == rowhit
        def plane(v):
            return jnp.max(jnp.where(rowsel, v, -1e30), axis=0, keepdims=True)
        pbx1 = plane(x1)
        pby1 = plane(y1)
        pbx2 = plane(x2)
        pby2 = plane(y2)
        lanewin = flat == idx
        def pick(v):
            return jnp.max(jnp.where(lanewin, v, -1e30), axis=1, keepdims=True)
        bx1 = pick(pbx1)
        by1 = pick(pby1)
        bx2 = pick(pbx2)
        by2 = pick(pby2)
        onehot = flat_iota_f == idx

        valid = (m > SCORE_THRESH).astype(jnp.float32)
        out_ref[pl.ds(i, 1), :] = jnp.where(
            lane == 0, bx1,
            jnp.where(lane == 1, by1,
            jnp.where(lane == 2, bx2,
            jnp.where(lane == 3, by2, m)))) * valid

        inter = (jnp.maximum(jnp.minimum(bx2, x2) - jnp.maximum(bx1, x1), 0.0)
                 * jnp.maximum(jnp.minimum(by2, y2) - jnp.maximum(by1, y1), 0.0))
        barea = (bx2 - bx1) * (by2 - by1)
        area = (x2 - x1) * (y2 - y1)
        iou = inter / (barea + area - inter + 1e-9)
        return jnp.where((iou > NMS_THRESH) | onehot, NEG, sc)

    def dstep(k, sc):
        sc = _one(2 * k, sc)
        return _one(2 * k + 1, sc)

    jax.lax.fori_loop(0, DET // 2, dstep, sc0)


@jax.jit
def kernel(boxes, scores):
    pad = ROWS * LANES - N
    x1 = jnp.pad(boxes[:, 0], (0, pad)).reshape(ROWS, LANES)
    y1 = jnp.pad(boxes[:, 1], (0, pad)).reshape(ROWS, LANES)
    x2 = jnp.pad(boxes[:, 2], (0, pad)).reshape(ROWS, LANES)
    y2 = jnp.pad(boxes[:, 3], (0, pad)).reshape(ROWS, LANES)
    s = jnp.pad(scores, (0, pad)).reshape(ROWS, LANES)

    out = pl.pallas_call(
        _nms_body,
        out_shape=jax.ShapeDtypeStruct((DET, LANES), jnp.float32),
        scratch_shapes=[pltpu.VMEM((ROWS, LANES), jnp.float32)],
    )(x1, y1, x2, y2, s)
    return out[:, :5]


# speedup vs baseline: 1.1465x; 1.0540x over previous
import jax
import jax.numpy as jnp
from jax.experimental import pallas as pl
from jax.experimental.pallas import tpu as pltpu

N = 20000
DET = 100
SCORE_THRESH = 0.05
NMS_THRESH = 0.5
NEG = -1e9

ROWS = 160
LANES = 128
BIGF = 3e7


def _nms_body(b_ref, s_ref, out_ref, sc_ref):
    lane = jax.lax.broadcasted_iota(jnp.int32, (1, LANES), 1)
    lane_f = lane.astype(jnp.float32)
    row160_f = jax.lax.broadcasted_iota(jnp.int32, (ROWS, 1), 0).astype(
        jnp.float32)
    flat_iota_f = (
        jax.lax.broadcasted_iota(jnp.int32, (ROWS, LANES), 0) * LANES
        + jax.lax.broadcasted_iota(jnp.int32, (ROWS, LANES), 1)
    ).astype(jnp.float32)

    sc0 = jnp.where(s_ref[...] > SCORE_THRESH, s_ref[...], NEG)

    def _one(i, sc):
        x1 = b_ref[0]
        y1 = b_ref[1]
        x2 = b_ref[2]
        y2 = b_ref[3]

        pls = jnp.max(sc, axis=0, keepdims=True)
        rowhit = jnp.min(jnp.where(sc == pls, row160_f, BIGF), axis=0,
                         keepdims=True)
        flat = rowhit * LANES + lane_f
        m = jnp.max(pls, axis=1, keepdims=True)
        idx = jnp.min(jnp.where(pls == m, flat, BIGF), axis=1,
                      keepdims=True)
        rowsel = row160_f == rowhit
        def plane(v):
            return jnp.max(jnp.where(rowsel, v, -1e30), axis=0, keepdims=True)
        pbx1 = plane(x1)
        pby1 = plane(y1)
        pbx2 = plane(x2)
        pby2 = plane(y2)
        lanewin = flat == idx
        def pick(v):
            return jnp.max(jnp.where(lanewin, v, -1e30), axis=1, keepdims=True)
        bx1 = pick(pbx1)
        by1 = pick(pby1)
        bx2 = pick(pbx2)
        by2 = pick(pby2)
        onehot = flat_iota_f == idx

        valid = (m > SCORE_THRESH).astype(jnp.float32)
        out_ref[pl.ds(i, 1), :] = jnp.where(
            lane == 0, bx1,
            jnp.where(lane == 1, by1,
            jnp.where(lane == 2, bx2,
            jnp.where(lane == 3, by2, m)))) * valid

        inter = (jnp.maximum(jnp.minimum(bx2, x2) - jnp.maximum(bx1, x1), 0.0)
                 * jnp.maximum(jnp.minimum(by2, y2) - jnp.maximum(by1, y1), 0.0))
        barea = (bx2 - bx1) * (by2 - by1)
        area = (x2 - x1) * (y2 - y1)
        iou = inter / (barea + area - inter + 1e-9)
        return jnp.where((iou > NMS_THRESH) | onehot, NEG, sc)

    def dstep(k, sc):
        sc = _one(2 * k, sc)
        return _one(2 * k + 1, sc)

    jax.lax.fori_loop(0, DET // 2, dstep, sc0)


@jax.jit
def kernel(boxes, scores):
    pad = ROWS * LANES - N
    b = jnp.pad(boxes.T, ((0, 0), (0, pad))).reshape(4, ROWS, LANES)
    s = jnp.pad(scores, (0, pad)).reshape(ROWS, LANES)

    out = pl.pallas_call(
        _nms_body,
        out_shape=jax.ShapeDtypeStruct((DET, LANES), jnp.float32),
        scratch_shapes=[pltpu.VMEM((ROWS, LANES), jnp.float32)],
    )(b, s)
    return out[:, :5]
